# Initial kernel scaffold; baseline (speedup 1.0000x reference)
import functools

import jax
import jax.numpy as jnp
from jax import lax
from jax.experimental import pallas as pl
from jax.experimental.pallas import tpu as pltpu
from jax.experimental.pallas import tpu_sc as plsc

N = 10000
E = 320000
D = 128

NC = 2
NS = 16
L = 16
NW = NC * NS
EPW = E // NW
K = 80
NCHUNK = EPW // K
RPT = N // NS
ZR = 125

BM = 1000
NB = N // BM


def _mm_body(x_ref, w_ref, ws_ref, b_ref, sup_ref, self_ref):
    xb = x_ref[...]
    sup_ref[...] = jnp.dot(xb, w_ref[...], preferred_element_type=jnp.float32)
    self_ref[...] = (
        jnp.dot(xb, ws_ref[...], preferred_element_type=jnp.float32) + b_ref[...]
    )


def _matmuls(x, weight, self_weight, bias):
    return pl.pallas_call(
        _mm_body,
        grid=(NB,),
        in_specs=[
            pl.BlockSpec((BM, D), lambda i: (i, 0)),
            pl.BlockSpec((D, D), lambda i: (0, 0)),
            pl.BlockSpec((D, D), lambda i: (0, 0)),
            pl.BlockSpec((1, D), lambda i: (0, 0)),
        ],
        out_specs=[
            pl.BlockSpec((BM, D), lambda i: (i, 0)),
            pl.BlockSpec((BM, D), lambda i: (i, 0)),
        ],
        out_shape=[
            jax.ShapeDtypeStruct((N, D), jnp.float32),
            jax.ShapeDtypeStruct((N, D), jnp.float32),
        ],
    )(x, weight, self_weight, bias.reshape(1, D))


def _agg_body(sup_hbm, src_hbm, dst_hbm, w_hbm, out_hbm,
              src_v, dst_v, w_v, rows_v, zero_v, acc_sh, sem):
    cid = lax.axis_index("c")
    sid = lax.axis_index("s")
    wid = sid * NC + cid
    base = wid * EPW

    def _zrow(r, _):
        for j in range(D // L):
            zero_v[r, pl.ds(j * L, L)] = jnp.zeros((L,), jnp.float32)
        return 0
    lax.fori_loop(0, ZR, _zrow, 0)
    for k in range(RPT // ZR):
        pltpu.sync_copy(zero_v, acc_sh.at[pl.ds(sid * RPT + k * ZR, ZR)])
    plsc.subcore_barrier()

    def _chunk(t, _):
        b = base + t * K
        pltpu.sync_copy(src_hbm.at[pl.ds(b, K)], src_v)
        pltpu.sync_copy(dst_hbm.at[pl.ds(b, K)], dst_v)
        pltpu.sync_copy(w_hbm.at[pl.ds(b, K)], w_v)
        pltpu.async_copy(sup_hbm.at[src_v], rows_v, sem).wait()

        def _scale(e, _):
            w16 = plsc.load_gather(w_v, [jnp.full((L,), e, jnp.int32)])
            for j in range(D // L):
                rows_v[e, pl.ds(j * L, L)] = rows_v[e, pl.ds(j * L, L)] * w16
            return 0
        lax.fori_loop(0, K, _scale, 0)

        pltpu.sync_copy(rows_v, acc_sh.at[dst_v], add=True)
        return 0
    lax.fori_loop(0, NCHUNK, _chunk, 0)
    plsc.subcore_barrier()

    for k in range(RPT // ZR):
        off = sid * RPT + k * ZR
        pltpu.sync_copy(acc_sh.at[pl.ds(off, ZR)], out_hbm.at[cid, pl.ds(off, ZR)])


def _aggregate(support, src, dst, edge_weight):
    mesh = plsc.VectorSubcoreMesh(core_axis_name="c", subcore_axis_name="s")
    f = functools.partial(
        pl.kernel,
        mesh=mesh,
        out_type=jax.ShapeDtypeStruct((NC, N, D), jnp.float32),
        scratch_types=[
            pltpu.VMEM((K,), jnp.int32),
            pltpu.VMEM((K,), jnp.int32),
            pltpu.VMEM((K,), jnp.float32),
            pltpu.VMEM((K, D), jnp.float32),
            pltpu.VMEM((ZR, D), jnp.float32),
            pltpu.VMEM_SHARED((N, D), jnp.float32),
            pltpu.SemaphoreType.DMA,
        ],
    )(_agg_body)
    return f(support, src, dst, edge_weight)


def _comb_body(p0_ref, p1_ref, s_ref, pre_ref, st_ref):
    i = pl.program_id(0)
    v = p0_ref[...] + p1_ref[...] + s_ref[...]
    pre_ref[...] = v
    cs = jnp.sum(v, axis=0, keepdims=True)
    cs2 = jnp.sum(v * v, axis=0, keepdims=True)
    st = jnp.concatenate([cs, cs2, jnp.zeros((6, D), jnp.float32)], axis=0)

    @pl.when(i == 0)
    def _():
        st_ref[...] = st

    @pl.when(i > 0)
    def _():
        st_ref[...] += st


def _combine(p0, p1, self_pre):
    return pl.pallas_call(
        _comb_body,
        grid=(NB,),
        in_specs=[
            pl.BlockSpec((BM, D), lambda i: (i, 0)),
            pl.BlockSpec((BM, D), lambda i: (i, 0)),
            pl.BlockSpec((BM, D), lambda i: (i, 0)),
        ],
        out_specs=[
            pl.BlockSpec((BM, D), lambda i: (i, 0)),
            pl.BlockSpec((8, D), lambda i: (0, 0)),
        ],
        out_shape=[
            jax.ShapeDtypeStruct((N, D), jnp.float32),
            jax.ShapeDtypeStruct((8, D), jnp.float32),
        ],
    )(p0, p1, self_pre)


def _bn_body(pre_ref, st_ref, g_ref, b_ref, o_ref):
    s = st_ref[0:1, :]
    s2 = st_ref[1:2, :]
    mean = s / N
    var = s2 / N - mean * mean
    rstd = lax.rsqrt(var + 1e-5)
    o_ref[...] = (pre_ref[...] - mean) * (rstd * g_ref[...]) + b_ref[...]


def _batchnorm(pre, stats, gamma, beta):
    return pl.pallas_call(
        _bn_body,
        grid=(NB,),
        in_specs=[
            pl.BlockSpec((BM, D), lambda i: (i, 0)),
            pl.BlockSpec((8, D), lambda i: (0, 0)),
            pl.BlockSpec((1, D), lambda i: (0, 0)),
            pl.BlockSpec((1, D), lambda i: (0, 0)),
        ],
        out_specs=pl.BlockSpec((BM, D), lambda i: (i, 0)),
        out_shape=jax.ShapeDtypeStruct((N, D), jnp.float32),
    )(pre, stats, gamma.reshape(1, D), beta.reshape(1, D))


def kernel(x, edge_weight, weight, self_weight, bias, gamma, beta, edge_index):
    support, self_pre = _matmuls(x, weight, self_weight, bias)
    dst = edge_index[0]
    src = edge_index[1]
    partials = _aggregate(support, src, dst, edge_weight)
    pre, stats = _combine(partials[0], partials[1], self_pre)
    return _batchnorm(pre, stats, gamma, beta)


# speedup vs baseline: 4.1693x; 4.1693x over previous
import functools

import jax
import jax.numpy as jnp
from jax import lax
from jax.experimental import pallas as pl
from jax.experimental.pallas import tpu as pltpu
from jax.experimental.pallas import tpu_sc as plsc

N = 10000
E = 320000
D = 128

NC = 2
NS = 16
L = 16
NW = NC * NS
EPW = E // NW
K = 80
NCHUNK = EPW // K
NP = 10240
RPT = NP // NS
ZR = 128

BM = 1000
NB = N // BM


def _mm_body(x_ref, w_ref, ws_ref, b_ref, sup_ref, self_ref):
    xb = x_ref[...]
    sup_ref[...] = jnp.dot(xb, w_ref[...], preferred_element_type=jnp.float32)
    self_ref[...] = (
        jnp.dot(xb, ws_ref[...], preferred_element_type=jnp.float32) + b_ref[...]
    )


def _matmuls(x, weight, self_weight, bias):
    return pl.pallas_call(
        _mm_body,
        grid=(NB,),
        in_specs=[
            pl.BlockSpec((BM, D), lambda i: (i, 0)),
            pl.BlockSpec((D, D), lambda i: (0, 0)),
            pl.BlockSpec((D, D), lambda i: (0, 0)),
            pl.BlockSpec((1, D), lambda i: (0, 0)),
        ],
        out_specs=[
            pl.BlockSpec((BM, D), lambda i: (i, 0)),
            pl.BlockSpec((BM, D), lambda i: (i, 0)),
        ],
        out_shape=[
            jax.ShapeDtypeStruct((N, D), jnp.float32),
            jax.ShapeDtypeStruct((N, D), jnp.float32),
        ],
    )(x, weight, self_weight, bias.reshape(1, D))


def _agg_body(sup_hbm, src_hbm, dst_hbm, w_hbm, out_hbm,
              src_v, dst_v, w_v, rows_v, zero_v, acc_sh, sem):
    cid = lax.axis_index("c")
    sid = lax.axis_index("s")
    wid = sid * NC + cid
    base = wid * EPW

    def _zrow(r, _):
        for j in range(D // L):
            zero_v[r, pl.ds(j * L, L)] = jnp.zeros((L,), jnp.float32)
        return 0
    lax.fori_loop(0, ZR, _zrow, 0)
    for k in range(RPT // ZR):
        pltpu.sync_copy(zero_v, acc_sh.at[pl.ds(sid * RPT + k * ZR, ZR)])
    plsc.subcore_barrier()

    def _chunk(t, _):
        b = base + t * K
        pltpu.sync_copy(src_hbm.at[pl.ds(b, K)], src_v)
        pltpu.sync_copy(dst_hbm.at[pl.ds(b, K)], dst_v)
        pltpu.sync_copy(w_hbm.at[pl.ds(b, K)], w_v)
        pltpu.async_copy(sup_hbm.at[src_v], rows_v, sem).wait()

        def _scale(g, _):
            w16 = w_v[pl.ds(g * L, L)]
            for i in range(L):
                wi = jnp.full((L,), w16[i], jnp.float32)
                e = g * L + i
                for j in range(D // L):
                    rows_v[e, pl.ds(j * L, L)] = rows_v[e, pl.ds(j * L, L)] * wi
            return 0
        lax.fori_loop(0, K // L, _scale, 0)

        pltpu.sync_copy(rows_v, acc_sh.at[dst_v], add=True)
        return 0
    lax.fori_loop(0, NCHUNK, _chunk, 0)
    plsc.subcore_barrier()

    for k in range(RPT // ZR):
        off = sid * RPT + k * ZR
        pltpu.sync_copy(acc_sh.at[pl.ds(off, ZR)], out_hbm.at[cid, pl.ds(off, ZR)])


def _aggregate(support, src, dst, edge_weight):
    mesh = plsc.VectorSubcoreMesh(core_axis_name="c", subcore_axis_name="s")
    f = functools.partial(
        pl.kernel,
        mesh=mesh,
        out_type=jax.ShapeDtypeStruct((NC, NP, D), jnp.float32),
        scratch_types=[
            pltpu.VMEM((K,), jnp.int32),
            pltpu.VMEM((K,), jnp.int32),
            pltpu.VMEM((K,), jnp.float32),
            pltpu.VMEM((K, D), jnp.float32),
            pltpu.VMEM((ZR, D), jnp.float32),
            pltpu.VMEM_SHARED((NP, D), jnp.float32),
            pltpu.SemaphoreType.DMA,
        ],
    )(_agg_body)
    return f(support, src, dst, edge_weight)


def _comb_body(p0_ref, p1_ref, s_ref, pre_ref, st_ref):
    i = pl.program_id(0)
    v = p0_ref[...] + p1_ref[...] + s_ref[...]
    pre_ref[...] = v
    cs = jnp.sum(v, axis=0, keepdims=True)
    cs2 = jnp.sum(v * v, axis=0, keepdims=True)
    st = jnp.concatenate([cs, cs2, jnp.zeros((6, D), jnp.float32)], axis=0)

    @pl.when(i == 0)
    def _():
        st_ref[...] = st

    @pl.when(i > 0)
    def _():
        st_ref[...] += st


def _combine(p0, p1, self_pre):
    return pl.pallas_call(
        _comb_body,
        grid=(NB,),
        in_specs=[
            pl.BlockSpec((BM, D), lambda i: (i, 0)),
            pl.BlockSpec((BM, D), lambda i: (i, 0)),
            pl.BlockSpec((BM, D), lambda i: (i, 0)),
        ],
        out_specs=[
            pl.BlockSpec((BM, D), lambda i: (i, 0)),
            pl.BlockSpec((8, D), lambda i: (0, 0)),
        ],
        out_shape=[
            jax.ShapeDtypeStruct((N, D), jnp.float32),
            jax.ShapeDtypeStruct((8, D), jnp.float32),
        ],
    )(p0, p1, self_pre)


def _bn_body(pre_ref, st_ref, g_ref, b_ref, o_ref):
    s = st_ref[0:1, :]
    s2 = st_ref[1:2, :]
    mean = s / N
    var = s2 / N - mean * mean
    rstd = lax.rsqrt(var + 1e-5)
    o_ref[...] = (pre_ref[...] - mean) * (rstd * g_ref[...]) + b_ref[...]


def _batchnorm(pre, stats, gamma, beta):
    return pl.pallas_call(
        _bn_body,
        grid=(NB,),
        in_specs=[
            pl.BlockSpec((BM, D), lambda i: (i, 0)),
            pl.BlockSpec((8, D), lambda i: (0, 0)),
            pl.BlockSpec((1, D), lambda i: (0, 0)),
            pl.BlockSpec((1, D), lambda i: (0, 0)),
        ],
        out_specs=pl.BlockSpec((BM, D), lambda i: (i, 0)),
        out_shape=jax.ShapeDtypeStruct((N, D), jnp.float32),
    )(pre, stats, gamma.reshape(1, D), beta.reshape(1, D))


def kernel(x, edge_weight, weight, self_weight, bias, gamma, beta, edge_index):
    support, self_pre = _matmuls(x, weight, self_weight, bias)
    dst = edge_index[0]
    src = edge_index[1]
    partials = _aggregate(support, src, dst, edge_weight)
    pre, stats = _combine(partials[0], partials[1], self_pre)
    return _batchnorm(pre, stats, gamma, beta)


# speedup vs baseline: 9.3496x; 2.2425x over previous
import functools

import jax
import jax.numpy as jnp
from jax import lax
from jax.experimental import pallas as pl
from jax.experimental.pallas import tpu as pltpu
from jax.experimental.pallas import tpu_sc as plsc

N = 10000
E = 320000
D = 128

NC = 2
NS = 16
L = 16
NW = NC * NS
EPW = E // NW
K = 80
NCHUNK = EPW // K
NP = 10240
RPT = NP // NS
ZR = 128

BM = 1000
NB = N // BM


def _mm_body(x_ref, w_ref, ws_ref, b_ref, sup_ref, self_ref):
    xb = x_ref[...]
    sup_ref[...] = jnp.dot(xb, w_ref[...], preferred_element_type=jnp.float32)
    self_ref[...] = (
        jnp.dot(xb, ws_ref[...], preferred_element_type=jnp.float32) + b_ref[...]
    )


def _matmuls(x, weight, self_weight, bias):
    return pl.pallas_call(
        _mm_body,
        grid=(NB,),
        in_specs=[
            pl.BlockSpec((BM, D), lambda i: (i, 0)),
            pl.BlockSpec((D, D), lambda i: (0, 0)),
            pl.BlockSpec((D, D), lambda i: (0, 0)),
            pl.BlockSpec((1, D), lambda i: (0, 0)),
        ],
        out_specs=[
            pl.BlockSpec((BM, D), lambda i: (i, 0)),
            pl.BlockSpec((BM, D), lambda i: (i, 0)),
        ],
        out_shape=[
            jax.ShapeDtypeStruct((N, D), jnp.float32),
            jax.ShapeDtypeStruct((N, D), jnp.float32),
        ],
    )(x, weight, self_weight, bias.reshape(1, D))


def _agg_body(sup_hbm, packed_hbm, w_hbm, out_hbm,
              packed_all, w_all, src_va, src_vb, dst_v, rows_a, rows_b,
              acc_sh, sem_a, sem_b):
    cid = lax.axis_index("c")
    sid = lax.axis_index("s")
    wid = sid * NC + cid

    pltpu.sync_copy(packed_hbm.at[wid], packed_all)
    pltpu.sync_copy(w_hbm.at[wid], w_all)

    def _zrow(r, _):
        for j in range(D // L):
            rows_a[r, pl.ds(j * L, L)] = jnp.zeros((L,), jnp.float32)
        return 0
    lax.fori_loop(0, K, _zrow, 0)
    for k in range(RPT // K):
        pltpu.sync_copy(rows_a, acc_sh.at[pl.ds(sid * RPT + k * K, K)])
    plsc.subcore_barrier()

    def _unpack_src(t, sbuf):
        def _g(g, _):
            p = packed_all[pl.ds(t * K + g * L, L)]
            sbuf[pl.ds(g * L, L)] = jnp.bitwise_and(p, 16383)
            return 0
        lax.fori_loop(0, K // L, _g, 0)

    def _unpack_dst(t):
        def _g(g, _):
            p = packed_all[pl.ds(t * K + g * L, L)]
            dst_v[pl.ds(g * L, L)] = lax.shift_right_logical(p, 14)
            return 0
        lax.fori_loop(0, K // L, _g, 0)

    def _scale(rows_v, t):
        def _body(g, _):
            w16 = w_all[pl.ds(t * K + g * L, L)]
            for i in range(L):
                wi = jnp.full((L,), w16[i], jnp.float32)
                e = g * L + i
                for j in range(D // L):
                    rows_v[e, pl.ds(j * L, L)] = rows_v[e, pl.ds(j * L, L)] * wi
            return 0
        lax.fori_loop(0, K // L, _body, 0)

    def _issue(src_v, rows_v, sem):
        pltpu.async_copy(sup_hbm.at[src_v], rows_v, sem)

    def _wait(src_v, rows_v, sem):
        pltpu.make_async_copy(sup_hbm.at[src_v], rows_v, sem).wait()

    def _scatter(t, rows_v):
        _unpack_dst(t)
        pltpu.sync_copy(rows_v, acc_sh.at[dst_v], add=True)

    _unpack_src(0, src_va)
    _issue(src_va, rows_a, sem_a)

    def _pair(u, _):
        t_a = 2 * u
        t_b = t_a + 1
        _unpack_src(t_b, src_vb)
        _issue(src_vb, rows_b, sem_b)
        _wait(src_va, rows_a, sem_a)
        _scale(rows_a, t_a)
        _scatter(t_a, rows_a)
        _unpack_src(t_a + 2, src_va)
        _issue(src_va, rows_a, sem_a)
        _wait(src_vb, rows_b, sem_b)
        _scale(rows_b, t_b)
        _scatter(t_b, rows_b)
        return 0
    lax.fori_loop(0, (NCHUNK - 1) // 2, _pair, 0)
    t_last = NCHUNK - 1
    _wait(src_va, rows_a, sem_a)
    _scale(rows_a, t_last)
    _scatter(t_last, rows_a)
    plsc.subcore_barrier()

    for k in range(RPT // ZR):
        off = sid * RPT + k * ZR
        pltpu.sync_copy(acc_sh.at[pl.ds(off, ZR)], out_hbm.at[cid, pl.ds(off, ZR)])


def _aggregate(support, src, dst, edge_weight):
    mesh = plsc.VectorSubcoreMesh(core_axis_name="c", subcore_axis_name="s")
    f = functools.partial(
        pl.kernel,
        mesh=mesh,
        out_type=jax.ShapeDtypeStruct((NC, NP, D), jnp.float32),
        scratch_types=[
            pltpu.VMEM((EPW,), jnp.int32),
            pltpu.VMEM((EPW,), jnp.float32),
            pltpu.VMEM((K,), jnp.int32),
            pltpu.VMEM((K,), jnp.int32),
            pltpu.VMEM((K,), jnp.int32),
            pltpu.VMEM((K, D), jnp.float32),
            pltpu.VMEM((K, D), jnp.float32),
            pltpu.VMEM_SHARED((NP, D), jnp.float32),
            pltpu.SemaphoreType.DMA,
            pltpu.SemaphoreType.DMA,
        ],
    )(_agg_body)
    packed = jnp.bitwise_or(jnp.left_shift(dst, 14), src).reshape(NW, EPW)
    w2 = edge_weight.reshape(NW, EPW)
    return f(support, packed, w2)


def _comb_body(p0_ref, p1_ref, s_ref, pre_ref, st_ref):
    i = pl.program_id(0)
    v = p0_ref[...] + p1_ref[...] + s_ref[...]
    pre_ref[...] = v
    cs = jnp.sum(v, axis=0, keepdims=True)
    cs2 = jnp.sum(v * v, axis=0, keepdims=True)
    st = jnp.concatenate([cs, cs2, jnp.zeros((6, D), jnp.float32)], axis=0)

    @pl.when(i == 0)
    def _():
        st_ref[...] = st

    @pl.when(i > 0)
    def _():
        st_ref[...] += st


def _combine(p0, p1, self_pre):
    return pl.pallas_call(
        _comb_body,
        grid=(NB,),
        in_specs=[
            pl.BlockSpec((BM, D), lambda i: (i, 0)),
            pl.BlockSpec((BM, D), lambda i: (i, 0)),
            pl.BlockSpec((BM, D), lambda i: (i, 0)),
        ],
        out_specs=[
            pl.BlockSpec((BM, D), lambda i: (i, 0)),
            pl.BlockSpec((8, D), lambda i: (0, 0)),
        ],
        out_shape=[
            jax.ShapeDtypeStruct((N, D), jnp.float32),
            jax.ShapeDtypeStruct((8, D), jnp.float32),
        ],
    )(p0, p1, self_pre)


def _bn_body(pre_ref, st_ref, g_ref, b_ref, o_ref):
    s = st_ref[0:1, :]
    s2 = st_ref[1:2, :]
    mean = s / N
    var = s2 / N - mean * mean
    rstd = lax.rsqrt(var + 1e-5)
    o_ref[...] = (pre_ref[...] - mean) * (rstd * g_ref[...]) + b_ref[...]


def _batchnorm(pre, stats, gamma, beta):
    return pl.pallas_call(
        _bn_body,
        grid=(NB,),
        in_specs=[
            pl.BlockSpec((BM, D), lambda i: (i, 0)),
            pl.BlockSpec((8, D), lambda i: (0, 0)),
            pl.BlockSpec((1, D), lambda i: (0, 0)),
            pl.BlockSpec((1, D), lambda i: (0, 0)),
        ],
        out_specs=pl.BlockSpec((BM, D), lambda i: (i, 0)),
        out_shape=jax.ShapeDtypeStruct((N, D), jnp.float32),
    )(pre, stats, gamma.reshape(1, D), beta.reshape(1, D))


def kernel(x, edge_weight, weight, self_weight, bias, gamma, beta, edge_index):
    support, self_pre = _matmuls(x, weight, self_weight, bias)
    dst = edge_index[0]
    src = edge_index[1]
    partials = _aggregate(support, src, dst, edge_weight)
    pre, stats = _combine(partials[0], partials[1], self_pre)
    return _batchnorm(pre, stats, gamma, beta)


# speedup vs baseline: 9.3984x; 1.0052x over previous
import functools

import jax
import jax.numpy as jnp
from jax import lax
from jax.experimental import pallas as pl
from jax.experimental.pallas import tpu as pltpu
from jax.experimental.pallas import tpu_sc as plsc

N = 10000
E = 320000
D = 128

NC = 2
NS = 16
L = 16
NW = NC * NS
EPW = E // NW
K = 80
NCHUNK = EPW // K
NP = 10240
RPT = NP // NS
ZR = 128
WS = 96
WC = 48
WSTRIDE = 6144

BM = 1000
NB = N // BM


def _mm_body(x_ref, w_ref, sup_ref):
    sup_ref[...] = jnp.dot(
        x_ref[...], w_ref[...], preferred_element_type=jnp.float32
    )


def _support_mm(x, weight):
    return pl.pallas_call(
        _mm_body,
        grid=(NB,),
        in_specs=[
            pl.BlockSpec((BM, D), lambda i: (i, 0)),
            pl.BlockSpec((D, D), lambda i: (0, 0)),
        ],
        out_specs=pl.BlockSpec((BM, D), lambda i: (i, 0)),
        out_shape=jax.ShapeDtypeStruct((N, D), jnp.float32),
    )(x, weight)


def _agg_body(sup_hbm, packed_hbm, w_hbm, out_hbm,
              packed_all, w_stage, w_chunk,
              idx0, idx1, idx2, rows0, rows1, rows2,
              acc_sh, gsem0, gsem1, gsem2, ssem0, ssem1, ssem2):
    idx = (idx0, idx1, idx2)
    rows = (rows0, rows1, rows2)
    gsem = (gsem0, gsem1, gsem2)
    ssem = (ssem0, ssem1, ssem2)
    cid = lax.axis_index("c")
    sid = lax.axis_index("s")
    wid = sid * NC + cid

    pltpu.sync_copy(packed_hbm.at[wid], packed_all)
    pltpu.sync_copy(w_hbm.at[pl.ds(wid * WSTRIDE, WSTRIDE)], w_stage)

    def _zrow(r, _):
        for j in range(D // L):
            rows0[r, pl.ds(j * L, L)] = jnp.zeros((L,), jnp.float32)
        return 0
    lax.fori_loop(0, K, _zrow, 0)
    for k in range(RPT // K):
        pltpu.sync_copy(rows0, acc_sh.at[pl.ds(sid * RPT + k * K, K)])
    plsc.subcore_barrier()

    def _unpack_src(t, b):
        def _g(g, _):
            p = packed_all[pl.ds(t * K + g * L, L)]
            idx[b][pl.ds(g * L, L)] = jnp.bitwise_and(p, 16383)
            return 0
        lax.fori_loop(0, K // L, _g, 0)

    def _unpack_dst(t, b):
        def _g(g, _):
            p = packed_all[pl.ds(t * K + g * L, L)]
            idx[b][pl.ds(g * L, L)] = lax.shift_right_logical(p, 14)
            return 0
        lax.fori_loop(0, K // L, _g, 0)

    def _unpack_w(t):
        for q in range(3):
            v = w_stage[pl.ds(t * WC + L * q, L)]
            wa = lax.bitcast_convert_type(jnp.left_shift(v, 16), jnp.float32)
            w_chunk[pl.ds(32 * q, L)] = wa
            if q < 2:
                wb = lax.bitcast_convert_type(
                    jnp.bitwise_and(v, jnp.int32(-65536)), jnp.float32
                )
                w_chunk[pl.ds(32 * q + L, L)] = wb

    def _scale(b):
        rv = rows[b]

        def _body(g, _):
            w16 = w_chunk[pl.ds(g * L, L)]
            for i in range(L):
                wi = jnp.full((L,), w16[i], jnp.float32)
                e = g * L + i
                for j in range(D // L):
                    rv[e, pl.ds(j * L, L)] = rv[e, pl.ds(j * L, L)] * wi
            return 0
        lax.fori_loop(0, K // L, _body, 0)

    def _issue_gather(t, b):
        _unpack_src(t, b)
        pltpu.async_copy(sup_hbm.at[idx[b]], rows[b], gsem[b])

    def _wait_gather(b):
        pltpu.make_async_copy(sup_hbm.at[idx[b]], rows[b], gsem[b]).wait()

    def _issue_scatter(t, b):
        _unpack_dst(t, b)
        pltpu.async_copy(rows[b], acc_sh.at[idx[b]], ssem[b], add=True)

    def _wait_scatter(b):
        pltpu.make_async_copy(rows[b], acc_sh.at[idx[b]], ssem[b]).wait()

    _issue_gather(0, 0)
    _issue_gather(1, 1)

    def _steady(u, _):
        for i in range(3):
            c = 3 * u + i
            b = i
            bn = (i + 2) % 3
            _wait_gather(b)
            _unpack_w(c)
            _scale(b)
            _issue_scatter(c, b)

            @pl.when(c >= 1)
            def _():
                _wait_scatter(bn)
            _issue_gather(c + 2, bn)
        return 0
    lax.fori_loop(0, (NCHUNK - 2) // 3, _steady, 0)

    for c in (NCHUNK - 2, NCHUNK - 1):
        b = c % 3
        _wait_gather(b)
        _unpack_w(c)
        _scale(b)
        _issue_scatter(c, b)
    for b in ((NCHUNK - 3) % 3, (NCHUNK - 2) % 3, (NCHUNK - 1) % 3):
        _wait_scatter(b)
    plsc.subcore_barrier()

    for k in range(RPT // ZR):
        off = sid * RPT + k * ZR
        pltpu.sync_copy(acc_sh.at[pl.ds(off, ZR)], out_hbm.at[cid, pl.ds(off, ZR)])


def _aggregate(support, src, dst, edge_weight):
    mesh = plsc.VectorSubcoreMesh(core_axis_name="c", subcore_axis_name="s")
    f = functools.partial(
        pl.kernel,
        mesh=mesh,
        out_type=jax.ShapeDtypeStruct((NC, NP, D), jnp.float32),
        scratch_types=[
            pltpu.VMEM((EPW,), jnp.int32),
            pltpu.VMEM((WSTRIDE,), jnp.int32),
            pltpu.VMEM((WS,), jnp.float32),
            pltpu.VMEM((K,), jnp.int32),
            pltpu.VMEM((K,), jnp.int32),
            pltpu.VMEM((K,), jnp.int32),
            pltpu.VMEM((K, D), jnp.float32),
            pltpu.VMEM((K, D), jnp.float32),
            pltpu.VMEM((K, D), jnp.float32),
            pltpu.VMEM_SHARED((NP, D), jnp.float32),
            pltpu.SemaphoreType.DMA,
            pltpu.SemaphoreType.DMA,
            pltpu.SemaphoreType.DMA,
            pltpu.SemaphoreType.DMA,
            pltpu.SemaphoreType.DMA,
            pltpu.SemaphoreType.DMA,
        ],
    )(_agg_body)
    packed = jnp.bitwise_or(jnp.left_shift(dst, 14), src).reshape(NW, EPW)
    wg = edge_weight.reshape(NW, NCHUNK, 5, L)
    wg = jnp.concatenate(
        [wg, jnp.zeros((NW, NCHUNK, 1, L), jnp.float32)], axis=2
    )
    wu = lax.bitcast_convert_type(
        wg.astype(jnp.bfloat16).reshape(NW, NCHUNK, 3, 2, L), jnp.uint16
    ).astype(jnp.uint32)
    wi = jnp.bitwise_or(jnp.left_shift(wu[:, :, :, 1, :], 16), wu[:, :, :, 0, :])
    w_st = lax.bitcast_convert_type(wi, jnp.int32).reshape(NW, NCHUNK * WC)
    w_st = jnp.pad(w_st, ((0, 0), (0, WSTRIDE - NCHUNK * WC)))
    w_st = w_st.reshape(NW * WSTRIDE)
    return f(support, packed, w_st)


def _comb_body(p0_ref, p1_ref, x_ref, ws_ref, b_ref, pre_ref, st_ref):
    i = pl.program_id(0)
    v = p0_ref[...] + p1_ref[...] + b_ref[...] + jnp.dot(
        x_ref[...], ws_ref[...], preferred_element_type=jnp.float32
    )
    pre_ref[...] = v
    cs = jnp.sum(v, axis=0, keepdims=True)
    cs2 = jnp.sum(v * v, axis=0, keepdims=True)
    st = jnp.concatenate([cs, cs2, jnp.zeros((6, D), jnp.float32)], axis=0)

    @pl.when(i == 0)
    def _():
        st_ref[...] = st

    @pl.when(i > 0)
    def _():
        st_ref[...] += st


def _combine(p0, p1, x, self_weight, bias):
    return pl.pallas_call(
        _comb_body,
        grid=(NB,),
        in_specs=[
            pl.BlockSpec((BM, D), lambda i: (i, 0)),
            pl.BlockSpec((BM, D), lambda i: (i, 0)),
            pl.BlockSpec((BM, D), lambda i: (i, 0)),
            pl.BlockSpec((D, D), lambda i: (0, 0)),
            pl.BlockSpec((1, D), lambda i: (0, 0)),
        ],
        out_specs=[
            pl.BlockSpec((BM, D), lambda i: (i, 0)),
            pl.BlockSpec((8, D), lambda i: (0, 0)),
        ],
        out_shape=[
            jax.ShapeDtypeStruct((N, D), jnp.float32),
            jax.ShapeDtypeStruct((8, D), jnp.float32),
        ],
    )(p0, p1, x, self_weight, bias.reshape(1, D))


def _bn_body(pre_ref, st_ref, g_ref, b_ref, o_ref):
    s = st_ref[0:1, :]
    s2 = st_ref[1:2, :]
    mean = s / N
    var = s2 / N - mean * mean
    rstd = lax.rsqrt(var + 1e-5)
    o_ref[...] = (pre_ref[...] - mean) * (rstd * g_ref[...]) + b_ref[...]


def _batchnorm(pre, stats, gamma, beta):
    return pl.pallas_call(
        _bn_body,
        grid=(NB,),
        in_specs=[
            pl.BlockSpec((BM, D), lambda i: (i, 0)),
            pl.BlockSpec((8, D), lambda i: (0, 0)),
            pl.BlockSpec((1, D), lambda i: (0, 0)),
            pl.BlockSpec((1, D), lambda i: (0, 0)),
        ],
        out_specs=pl.BlockSpec((BM, D), lambda i: (i, 0)),
        out_shape=jax.ShapeDtypeStruct((N, D), jnp.float32),
    )(pre, stats, gamma.reshape(1, D), beta.reshape(1, D))


def kernel(x, edge_weight, weight, self_weight, bias, gamma, beta, edge_index):
    support = _support_mm(x, weight)
    dst = edge_index[0]
    src = edge_index[1]
    partials = _aggregate(support, src, dst, edge_weight)
    pre, stats = _combine(partials[0], partials[1], x, self_weight, bias)
    return _batchnorm(pre, stats, gamma, beta)
